# Initial kernel scaffold; baseline (speedup 1.0000x reference)
import jax
import jax.numpy as jnp
from jax.experimental import pallas as pl


def kernel(x, e, u, edge_index, batch, W1, b1, W2, b2, gamma, beta):
    raise NotImplementedError("write your pallas kernel here")


# speedup vs baseline: 13.4377x; 13.4377x over previous
import functools

import jax
import jax.numpy as jnp
from jax import lax
from jax.experimental import pallas as pl
from jax.experimental.pallas import tpu as pltpu
from jax.experimental.pallas import tpu_sc as plsc

N_NODES = 10000
N_EDGES = 320000
N_GRAPHS = 16
D_FEAT = 128
D_EDGE = 16
LATENT = 16

NC = 2
NS = 16
NW = NC * NS

STEP = 125
STEPS_TOTAL = N_EDGES // STEP
SPW = STEPS_TOTAL // NW
G = 8
NGRP = SPW // G


def _node_pre_body(x_ref, w_ref, u_ref, wu_ref, batch_ref, prer_ref, pres_ref):
    pre = jnp.dot(x_ref[...], w_ref[...], preferred_element_type=jnp.float32)
    uw = jnp.dot(u_ref[...], wu_ref[...], preferred_element_type=jnp.float32)
    onehot = (batch_ref[...] == lax.broadcasted_iota(jnp.int32, (1, N_GRAPHS), 1))
    ub = jnp.dot(onehot.astype(jnp.float32), uw, preferred_element_type=jnp.float32)
    prer_ref[...] = pre[:, :LATENT]
    pres_ref[...] = pre[:, LATENT:] + ub


def _node_pre(x, w_rs, u, w_u, batch2d):
    return pl.pallas_call(
        _node_pre_body,
        out_shape=(
            jax.ShapeDtypeStruct((N_NODES, LATENT), jnp.float32),
            jax.ShapeDtypeStruct((N_NODES, LATENT), jnp.float32),
        ),
    )(x, w_rs, u, w_u, batch2d)


def _gather_body(prer_hbm, pres_hbm, col_hbm, row_hbm, gr_hbm, gs_hbm,
                 colv, rowv, bufr, bufs, semr, sems):
    wid = lax.axis_index("s") * NC + lax.axis_index("c")
    base = wid * SPW
    pltpu.sync_copy(col_hbm.at[pl.ds(base, SPW)], colv)
    pltpu.sync_copy(row_hbm.at[pl.ds(base, SPW)], rowv)

    def grp_body(g, carry):
        cps = []
        for b in range(G):
            j = g * G + b
            cps.append(pltpu.async_copy(prer_hbm.at[colv.at[j]], bufr.at[b], semr))
            cps.append(pltpu.async_copy(pres_hbm.at[rowv.at[j]], bufs.at[b], sems))
        for cp in cps:
            cp.wait()
        pltpu.sync_copy(bufr, gr_hbm.at[pl.ds(base + g * G, G)])
        pltpu.sync_copy(bufs, gs_hbm.at[pl.ds(base + g * G, G)])
        return carry

    lax.fori_loop(0, NGRP, grp_body, 0)


def _gather(prer, pres, col2, row2):
    mesh = plsc.VectorSubcoreMesh(core_axis_name="c", subcore_axis_name="s")
    fn = pl.kernel(
        _gather_body,
        compiler_params=pltpu.CompilerParams(use_tc_tiling_on_sc=False),
        out_type=(
            jax.ShapeDtypeStruct((STEPS_TOTAL, STEP, LATENT), jnp.float32),
            jax.ShapeDtypeStruct((STEPS_TOTAL, STEP, LATENT), jnp.float32),
        ),
        mesh=mesh,
        scratch_types=[
            pltpu.VMEM((SPW, STEP), jnp.int32),
            pltpu.VMEM((SPW, STEP), jnp.int32),
            pltpu.VMEM((G, STEP, LATENT), jnp.float32),
            pltpu.VMEM((G, STEP, LATENT), jnp.float32),
            pltpu.SemaphoreType.DMA,
            pltpu.SemaphoreType.DMA,
        ],
    )
    return fn(prer, pres, col2, row2)


_BLK = 5000
_ROWS8 = N_EDGES // 8


def _mlp_body(e_ref, gr_ref, gs_ref, w1_ref, w2_ref, m_ref, bb_ref, out_ref):
    a = jnp.dot(e_ref[...], w1_ref[...], preferred_element_type=jnp.float32)
    h = jnp.maximum(a + gr_ref[...] + gs_ref[...] + bb_ref[0:1, :], 0.0)
    h2 = jnp.dot(h, w2_ref[...], preferred_element_type=jnp.float32) + bb_ref[1:2, :]
    h2 = jnp.maximum(h2, 0.0)
    mu = jnp.dot(h2, m_ref[...], preferred_element_type=jnp.float32)
    d = h2 - mu
    var = jnp.dot(d * d, m_ref[...], preferred_element_type=jnp.float32)
    out_ref[...] = d * lax.rsqrt(var + 1e-5) * bb_ref[2:3, :] + bb_ref[3:4, :]


def _mlp(e8, gr8, gs8, w1e_t, w2_t, m_t, bb):
    big = pl.BlockSpec((_BLK, 128), lambda i: (i, 0))
    small = pl.BlockSpec((128, 128), lambda i: (0, 0))
    return pl.pallas_call(
        _mlp_body,
        grid=(_ROWS8 // _BLK,),
        in_specs=[big, big, big, small, small, small,
                  pl.BlockSpec((4, 128), lambda i: (0, 0))],
        out_specs=big,
        out_shape=jax.ShapeDtypeStruct((_ROWS8, 128), jnp.float32),
    )(e8, gr8, gs8, w1e_t, w2_t, m_t, bb)


def kernel(x, e, u, edge_index, batch, W1, b1, W2, b2, gamma, beta):
    f32 = jnp.float32
    w_rs = jnp.concatenate([W1[D_EDGE:D_EDGE + D_FEAT],
                            W1[D_EDGE + D_FEAT:D_EDGE + 2 * D_FEAT]], axis=1)
    w_u = W1[D_EDGE + 2 * D_FEAT:]
    prer, pres = _node_pre(x, w_rs, u, w_u, batch.reshape(N_NODES, 1))

    col2 = edge_index[1].reshape(STEPS_TOTAL, STEP)
    row2 = edge_index[0].reshape(STEPS_TOTAL, STEP)
    gr3, gs3 = _gather(prer, pres, col2, row2)

    eye8 = jnp.eye(8, dtype=f32)
    w1e_t = jnp.kron(eye8, W1[:D_EDGE])
    w2_t = jnp.kron(eye8, W2)
    m_t = jnp.kron(eye8, jnp.full((LATENT, LATENT), 1.0 / LATENT, dtype=f32))
    bb = jnp.stack([jnp.tile(b1, 8), jnp.tile(b2, 8),
                    jnp.tile(gamma, 8), jnp.tile(beta, 8)])

    out8 = _mlp(e.reshape(_ROWS8, 128), gr3.reshape(_ROWS8, 128),
                gs3.reshape(_ROWS8, 128), w1e_t, w2_t, m_t, bb)
    return out8.reshape(N_EDGES, LATENT)


# speedup vs baseline: 14.1437x; 1.0525x over previous
import functools

import jax
import jax.numpy as jnp
from jax import lax
from jax.experimental import pallas as pl
from jax.experimental.pallas import tpu as pltpu
from jax.experimental.pallas import tpu_sc as plsc

N_NODES = 10000
N_EDGES = 320000
N_GRAPHS = 16
D_FEAT = 128
D_EDGE = 16
LATENT = 16

NC = 2
NS = 16
NW = NC * NS

EPW = N_EDGES // NW
STEP = 128
G = 6
NGRP = 13
TAIL = EPW - NGRP * G * STEP
GROUP_E = G * STEP


def _node_pre_body(x8_ref, wbr_ref, wbs_ref, u_ref, wu_ref, b8_ref, r_ref,
                   m8_ref, prer_ref, pres_ref):
    x8 = x8_ref[...]
    prer_ref[...] = jnp.dot(x8, wbr_ref[...], preferred_element_type=jnp.float32)
    uw = jnp.dot(u_ref[...], wu_ref[...], preferred_element_type=jnp.float32)
    uw8 = jnp.tile(uw, (8, 8)) * m8_ref[...]
    brep = jnp.dot(b8_ref[...].astype(jnp.float32), r_ref[...],
                   preferred_element_type=jnp.float32)
    g16 = (lax.broadcasted_iota(jnp.int32, (1, 128), 1) % 16).astype(jnp.float32)
    oh = (brep == g16).astype(jnp.float32)
    pres_ref[...] = (jnp.dot(x8, wbs_ref[...], preferred_element_type=jnp.float32)
                     + jnp.dot(oh, uw8, preferred_element_type=jnp.float32))


def _node_pre(x8, wbr, wbs, u, wu, b8, r, m8):
    return pl.pallas_call(
        _node_pre_body,
        out_shape=(
            jax.ShapeDtypeStruct((N_NODES // 8, 128), jnp.float32),
            jax.ShapeDtypeStruct((N_NODES // 8, 128), jnp.float32),
        ),
    )(x8, wbr, wbs, u, wu, b8, r, m8)


def _gather_body(prer_hbm, pres_hbm, col_hbm, row_hbm, gr_hbm, gs_hbm,
                 colv, rowv, bufr, bufs, tbr, tbs, semr, sems):
    wid = lax.axis_index("s") * NC + lax.axis_index("c")
    ebase = wid * EPW
    pltpu.sync_copy(col_hbm.at[pl.ds(ebase, EPW)], colv)
    pltpu.sync_copy(row_hbm.at[pl.ds(ebase, EPW)], rowv)

    def grp_body(g, carry):
        cps = []
        for b in range(G):
            o = g * GROUP_E + b * STEP
            d = pl.ds(b * STEP, STEP)
            cps.append(pltpu.async_copy(
                prer_hbm.at[colv.at[pl.ds(o, STEP)]], bufr.at[d], semr))
            cps.append(pltpu.async_copy(
                pres_hbm.at[rowv.at[pl.ds(o, STEP)]], bufs.at[d], sems))
        for cp in cps:
            cp.wait()
        pltpu.sync_copy(bufr, gr_hbm.at[pl.ds(ebase + g * GROUP_E, GROUP_E)])
        pltpu.sync_copy(bufs, gs_hbm.at[pl.ds(ebase + g * GROUP_E, GROUP_E)])
        return carry

    lax.fori_loop(0, NGRP, grp_body, 0)

    to = NGRP * GROUP_E
    cpr = pltpu.async_copy(prer_hbm.at[colv.at[pl.ds(to, TAIL)]], tbr, semr)
    cps = pltpu.async_copy(pres_hbm.at[rowv.at[pl.ds(to, TAIL)]], tbs, sems)
    cpr.wait()
    cps.wait()
    pltpu.sync_copy(tbr, gr_hbm.at[pl.ds(ebase + to, TAIL)])
    pltpu.sync_copy(tbs, gs_hbm.at[pl.ds(ebase + to, TAIL)])


def _gather(prer_p, pres_p, col1, row1):
    mesh = plsc.VectorSubcoreMesh(core_axis_name="c", subcore_axis_name="s")
    fn = pl.kernel(
        _gather_body,
        compiler_params=pltpu.CompilerParams(use_tc_tiling_on_sc=False),
        out_type=(
            jax.ShapeDtypeStruct((N_EDGES, LATENT), jnp.float32),
            jax.ShapeDtypeStruct((N_EDGES, LATENT), jnp.float32),
        ),
        mesh=mesh,
        scratch_types=[
            pltpu.VMEM((EPW,), jnp.int32),
            pltpu.VMEM((EPW,), jnp.int32),
            pltpu.VMEM((GROUP_E, LATENT), jnp.float32),
            pltpu.VMEM((GROUP_E, LATENT), jnp.float32),
            pltpu.VMEM((TAIL, LATENT), jnp.float32),
            pltpu.VMEM((TAIL, LATENT), jnp.float32),
            pltpu.SemaphoreType.DMA,
            pltpu.SemaphoreType.DMA,
        ],
    )
    prer = prer_p.reshape(N_NODES, LATENT)
    pres = pres_p.reshape(N_NODES, LATENT)
    return fn(prer, pres, col1, row1)


_BLK = 5000
_ROWS8 = N_EDGES // 8


def _mlp_body(e_ref, gr_ref, gs_ref, w1_ref, w2_ref, m_ref, bb_ref, out_ref):
    a = jnp.dot(e_ref[...], w1_ref[...], preferred_element_type=jnp.float32)
    h = jnp.maximum(a + gr_ref[...] + gs_ref[...] + bb_ref[0:1, :], 0.0)
    h2 = jnp.dot(h, w2_ref[...], preferred_element_type=jnp.float32) + bb_ref[1:2, :]
    h2 = jnp.maximum(h2, 0.0)
    mu = jnp.dot(h2, m_ref[...], preferred_element_type=jnp.float32)
    d = h2 - mu
    var = jnp.dot(d * d, m_ref[...], preferred_element_type=jnp.float32)
    out_ref[...] = d * lax.rsqrt(var + 1e-5) * bb_ref[2:3, :] + bb_ref[3:4, :]


def _mlp(e8, gr8, gs8, w1e_t, w2_t, m_t, bb):
    big = pl.BlockSpec((_BLK, 128), lambda i: (i, 0))
    small = pl.BlockSpec((128, 128), lambda i: (0, 0))
    return pl.pallas_call(
        _mlp_body,
        grid=(_ROWS8 // _BLK,),
        in_specs=[big, big, big, small, small, small,
                  pl.BlockSpec((4, 128), lambda i: (0, 0))],
        out_specs=big,
        out_shape=jax.ShapeDtypeStruct((_ROWS8, 128), jnp.float32),
    )(e8, gr8, gs8, w1e_t, w2_t, m_t, bb)


def kernel(x, e, u, edge_index, batch, W1, b1, W2, b2, gamma, beta):
    f32 = jnp.float32
    eye8 = jnp.eye(8, dtype=f32)
    w_r = W1[D_EDGE:D_EDGE + D_FEAT]
    w_s = W1[D_EDGE + D_FEAT:D_EDGE + 2 * D_FEAT]
    w_u = W1[D_EDGE + 2 * D_FEAT:]
    wbr = jnp.kron(eye8, w_r)
    wbs = jnp.kron(eye8, w_s)
    m8 = jnp.kron(eye8, jnp.ones((16, 16), f32))
    rmat = jnp.kron(eye8, jnp.ones((1, 16), f32))

    x8 = x.reshape(N_NODES // 8, 8 * D_FEAT)
    b8 = batch.reshape(N_NODES // 8, 8)
    prer_p, pres_p = _node_pre(x8, wbr, wbs, u, w_u, b8, rmat, m8)

    col1 = edge_index[1]
    row1 = edge_index[0]
    gr, gs = _gather(prer_p, pres_p, col1, row1)

    w1e_t = jnp.kron(eye8, W1[:D_EDGE])
    w2_t = jnp.kron(eye8, W2)
    m_t = jnp.kron(eye8, jnp.full((LATENT, LATENT), 1.0 / LATENT, dtype=f32))
    bb = jnp.stack([jnp.tile(b1, 8), jnp.tile(b2, 8),
                    jnp.tile(gamma, 8), jnp.tile(beta, 8)])

    out8 = _mlp(e.reshape(_ROWS8, 128), gr.reshape(_ROWS8, 128),
                gs.reshape(_ROWS8, 128), w1e_t, w2_t, m_t, bb)
    return out8.reshape(N_EDGES, LATENT)


# speedup vs baseline: 18.9227x; 1.3379x over previous
import functools

import jax
import jax.numpy as jnp
from jax import lax
from jax.experimental import pallas as pl
from jax.experimental.pallas import tpu as pltpu
from jax.experimental.pallas import tpu_sc as plsc

N_NODES = 10000
N_EDGES = 320000
N_GRAPHS = 16
D_FEAT = 128
D_EDGE = 16
LATENT = 16

NC = 2
NS = 16
NW = NC * NS
LANE = 16

EPW = N_EDGES // NW
STEP = 128
G = 6
NGRP = 13
GROUP_E = G * STEP
TAIL = EPW - NGRP * GROUP_E
VPG = GROUP_E // LANE


def _node_pre_body(x8_ref, wbr_ref, wbs_ref, u_ref, wu_ref, b8_ref, r_ref,
                   m8_ref, prer_ref, pres_ref):
    x8 = x8_ref[...]
    prer_ref[...] = jnp.dot(x8, wbr_ref[...], preferred_element_type=jnp.float32)
    uw = jnp.dot(u_ref[...], wu_ref[...], preferred_element_type=jnp.float32)
    uw8 = jnp.tile(uw, (8, 8)) * m8_ref[...]
    brep = jnp.dot(b8_ref[...].astype(jnp.float32), r_ref[...],
                   preferred_element_type=jnp.float32)
    g16 = (lax.broadcasted_iota(jnp.int32, (1, 128), 1) % 16).astype(jnp.float32)
    oh = (brep == g16).astype(jnp.float32)
    pres_ref[...] = (jnp.dot(x8, wbs_ref[...], preferred_element_type=jnp.float32)
                     + jnp.dot(oh, uw8, preferred_element_type=jnp.float32))


def _node_pre(x8, wbr, wbs, u, wu, b8, r, m8):
    return pl.pallas_call(
        _node_pre_body,
        out_shape=(
            jax.ShapeDtypeStruct((N_NODES // 8, 128), jnp.float32),
            jax.ShapeDtypeStruct((N_NODES // 8, 128), jnp.float32),
        ),
    )(x8, wbr, wbs, u, wu, b8, r, m8)


def _gather_body(prer_hbm, pres_hbm, col_hbm, row_hbm, gt_hbm,
                 colv, rowv, bufr, bufs, buft, semr, sems):
    wid = lax.axis_index("s") * NC + lax.axis_index("c")
    ebase = wid * EPW
    pltpu.sync_copy(col_hbm.at[pl.ds(ebase, EPW)], colv)
    pltpu.sync_copy(row_hbm.at[pl.ds(ebase, EPW)], rowv)
    lane = lax.broadcasted_iota(jnp.int32, (LANE,), 0)

    def transpose_group(n_vregs):
        def frow(f, carry):
            fcol = jnp.full((LANE,), f, jnp.int32)
            for b in range(n_vregs):
                ridx = lane + (LANE * b)
                v = (plsc.load_gather(bufr, [ridx, fcol])
                     + plsc.load_gather(bufs, [ridx, fcol]))
                buft[f, pl.ds(b * LANE, LANE)] = v
            return carry
        lax.fori_loop(0, LATENT, frow, 0)

    def grp_body(g, carry):
        cps = []
        for b in range(G):
            o = g * GROUP_E + b * STEP
            d = pl.ds(b * STEP, STEP)
            cps.append(pltpu.async_copy(
                prer_hbm.at[colv.at[pl.ds(o, STEP)]], bufr.at[d], semr))
            cps.append(pltpu.async_copy(
                pres_hbm.at[rowv.at[pl.ds(o, STEP)]], bufs.at[d], sems))
        for cp in cps:
            cp.wait()
        transpose_group(VPG)
        pltpu.sync_copy(buft,
                        gt_hbm.at[:, pl.ds(ebase + g * GROUP_E, GROUP_E)])
        return carry

    lax.fori_loop(0, NGRP, grp_body, 0)

    to = NGRP * GROUP_E
    cpr = pltpu.async_copy(prer_hbm.at[colv.at[pl.ds(to, TAIL)]],
                           bufr.at[pl.ds(0, TAIL)], semr)
    cps = pltpu.async_copy(pres_hbm.at[rowv.at[pl.ds(to, TAIL)]],
                           bufs.at[pl.ds(0, TAIL)], sems)
    cpr.wait()
    cps.wait()
    transpose_group(TAIL // LANE)
    pltpu.sync_copy(buft.at[:, pl.ds(0, TAIL)],
                    gt_hbm.at[:, pl.ds(ebase + to, TAIL)])


def _gather(prer_p, pres_p, col1, row1):
    mesh = plsc.VectorSubcoreMesh(core_axis_name="c", subcore_axis_name="s")
    fn = pl.kernel(
        _gather_body,
        compiler_params=pltpu.CompilerParams(use_tc_tiling_on_sc=False,
                                             needs_layout_passes=False),
        out_type=jax.ShapeDtypeStruct((LATENT, N_EDGES), jnp.float32),
        mesh=mesh,
        scratch_types=[
            pltpu.VMEM((EPW,), jnp.int32),
            pltpu.VMEM((EPW,), jnp.int32),
            pltpu.VMEM((GROUP_E, LATENT), jnp.float32),
            pltpu.VMEM((GROUP_E, LATENT), jnp.float32),
            pltpu.VMEM((LATENT, GROUP_E), jnp.float32),
            pltpu.SemaphoreType.DMA,
            pltpu.SemaphoreType.DMA,
        ],
    )
    prer = prer_p.reshape(N_NODES, LATENT)
    pres = pres_p.reshape(N_NODES, LATENT)
    return fn(prer, pres, col1, row1)


_BLK = 32000


def _mlp_body(et_ref, gt_ref, w1t_ref, w2t_ref, bb_ref, out_ref):
    t = jnp.dot(w1t_ref[...], et_ref[...], preferred_element_type=jnp.float32)
    h = jnp.maximum(t + gt_ref[...] + bb_ref[:, 0:1], 0.0)
    h2 = jnp.dot(w2t_ref[...], h, preferred_element_type=jnp.float32) + bb_ref[:, 1:2]
    h2 = jnp.maximum(h2, 0.0)
    mu = jnp.mean(h2, axis=0, keepdims=True)
    d = h2 - mu
    var = jnp.mean(d * d, axis=0, keepdims=True)
    out_ref[...] = d * lax.rsqrt(var + 1e-5) * bb_ref[:, 2:3] + bb_ref[:, 3:4]


def _mlp(et, gt, w1t, w2t, bb):
    big = pl.BlockSpec((LATENT, _BLK), lambda i: (0, i))
    return pl.pallas_call(
        _mlp_body,
        grid=(N_EDGES // _BLK,),
        in_specs=[big, big,
                  pl.BlockSpec((LATENT, LATENT), lambda i: (0, 0)),
                  pl.BlockSpec((LATENT, LATENT), lambda i: (0, 0)),
                  pl.BlockSpec((LATENT, 4), lambda i: (0, 0))],
        out_specs=big,
        out_shape=jax.ShapeDtypeStruct((LATENT, N_EDGES), jnp.float32),
    )(et, gt, w1t, w2t, bb)


def kernel(x, e, u, edge_index, batch, W1, b1, W2, b2, gamma, beta):
    f32 = jnp.float32
    eye8 = jnp.eye(8, dtype=f32)
    w_r = W1[D_EDGE:D_EDGE + D_FEAT]
    w_s = W1[D_EDGE + D_FEAT:D_EDGE + 2 * D_FEAT]
    w_u = W1[D_EDGE + 2 * D_FEAT:]
    wbr = jnp.kron(eye8, w_r)
    wbs = jnp.kron(eye8, w_s)
    m8 = jnp.kron(eye8, jnp.ones((16, 16), f32))
    rmat = jnp.kron(eye8, jnp.ones((1, 16), f32))

    x8 = x.reshape(N_NODES // 8, 8 * D_FEAT)
    b8 = batch.reshape(N_NODES // 8, 8)
    prer_p, pres_p = _node_pre(x8, wbr, wbs, u, w_u, b8, rmat, m8)

    gt = _gather(prer_p, pres_p, edge_index[1], edge_index[0])

    w1t = W1[:D_EDGE].T
    w2t = W2.T
    bb = jnp.stack([b1, b2, gamma, beta], axis=1)

    outt = _mlp(e.T, gt, w1t, w2t, bb)
    return outt.T


# speedup vs baseline: 21.5205x; 1.1373x over previous
import functools

import jax
import jax.numpy as jnp
from jax import lax
from jax.experimental import pallas as pl
from jax.experimental.pallas import tpu as pltpu
from jax.experimental.pallas import tpu_sc as plsc

N_NODES = 10000
N_EDGES = 320000
N_GRAPHS = 16
D_FEAT = 128
D_EDGE = 16
LATENT = 16

NC = 2
NS = 16
NW = NC * NS
LANE = 16

EPW = N_EDGES // NW
STEP = 128
G = 3
NGRP = 26
NPAIR = NGRP // 2
GROUP_E = G * STEP
TAIL = EPW - NGRP * GROUP_E
VPG = GROUP_E // LANE


def _node_pre_body(x8_ref, wbr_ref, wbs_ref, u_ref, wu_ref, b8_ref, r_ref,
                   m8_ref, prer_ref, pres_ref):
    x8 = x8_ref[...]
    prer_ref[...] = jnp.dot(x8, wbr_ref[...], preferred_element_type=jnp.float32)
    uw = jnp.dot(u_ref[...], wu_ref[...], preferred_element_type=jnp.float32)
    uw8 = jnp.tile(uw, (8, 8)) * m8_ref[...]
    brep = jnp.dot(b8_ref[...].astype(jnp.float32), r_ref[...],
                   preferred_element_type=jnp.float32)
    g16 = (lax.broadcasted_iota(jnp.int32, (1, 128), 1) % 16).astype(jnp.float32)
    oh = (brep == g16).astype(jnp.float32)
    pres_ref[...] = (jnp.dot(x8, wbs_ref[...], preferred_element_type=jnp.float32)
                     + jnp.dot(oh, uw8, preferred_element_type=jnp.float32))


def _node_pre(x8, wbr, wbs, u, wu, b8, r, m8):
    return pl.pallas_call(
        _node_pre_body,
        out_shape=(
            jax.ShapeDtypeStruct((N_NODES // 8, 128), jnp.float32),
            jax.ShapeDtypeStruct((N_NODES // 8, 128), jnp.float32),
        ),
    )(x8, wbr, wbs, u, wu, b8, r, m8)


def _gather_body(prer_hbm, pres_hbm, col_hbm, row_hbm, gt_hbm,
                 colv, rowv, bufr0, bufs0, bufr1, bufs1, buft0, buft1,
                 semr0, sems0, semr1, sems1, semw0, semw1):
    wid = lax.axis_index("s") * NC + lax.axis_index("c")
    ebase = wid * EPW
    pltpu.sync_copy(col_hbm.at[pl.ds(ebase, EPW)], colv)
    pltpu.sync_copy(row_hbm.at[pl.ds(ebase, EPW)], rowv)
    lane = lax.broadcasted_iota(jnp.int32, (LANE,), 0)

    def fire(g, bufr, bufs, semr, sems):
        for b in range(G):
            o = g * GROUP_E + b * STEP
            d = pl.ds(b * STEP, STEP)
            pltpu.async_copy(prer_hbm.at[colv.at[pl.ds(o, STEP)]],
                             bufr.at[d], semr)
            pltpu.async_copy(pres_hbm.at[rowv.at[pl.ds(o, STEP)]],
                             bufs.at[d], sems)

    def drain_gathers(bufr, bufs, semr, sems):
        idx0 = colv.at[pl.ds(0, STEP)]
        for b in range(G):
            d = pl.ds(b * STEP, STEP)
            pltpu.make_async_copy(prer_hbm.at[idx0], bufr.at[d], semr).wait()
            pltpu.make_async_copy(pres_hbm.at[idx0], bufs.at[d], sems).wait()

    def transpose(bufr, bufs, buft, n_vregs):
        def frow(f, carry):
            fcol = jnp.full((LANE,), f, jnp.int32)
            for b in range(n_vregs):
                ridx = lane + (LANE * b)
                v = (plsc.load_gather(bufr, [ridx, fcol])
                     + plsc.load_gather(bufs, [ridx, fcol]))
                buft[f, pl.ds(b * LANE, LANE)] = v
            return carry
        lax.fori_loop(0, LATENT, frow, 0)

    def write(buft, g, semw):
        pltpu.async_copy(buft,
                         gt_hbm.at[:, pl.ds(ebase + g * GROUP_E, GROUP_E)],
                         semw)

    def drain_write(buft, semw):
        pltpu.make_async_copy(buft, gt_hbm.at[:, pl.ds(ebase, GROUP_E)],
                              semw).wait()

    fire(0, bufr0, bufs0, semr0, sems0)

    def pair(p, carry):
        g0 = 2 * p
        fire(g0 + 1, bufr1, bufs1, semr1, sems1)
        drain_gathers(bufr0, bufs0, semr0, sems0)

        @pl.when(p > 0)
        def _():
            drain_write(buft0, semw0)

        transpose(bufr0, bufs0, buft0, VPG)
        write(buft0, g0, semw0)

        @pl.when(p < NPAIR - 1)
        def _():
            fire(g0 + 2, bufr0, bufs0, semr0, sems0)

        drain_gathers(bufr1, bufs1, semr1, sems1)

        @pl.when(p > 0)
        def _():
            drain_write(buft1, semw1)

        transpose(bufr1, bufs1, buft1, VPG)
        write(buft1, g0 + 1, semw1)
        return carry

    lax.fori_loop(0, NPAIR, pair, 0)
    drain_write(buft0, semw0)
    drain_write(buft1, semw1)

    to = NGRP * GROUP_E
    cpr = pltpu.async_copy(prer_hbm.at[colv.at[pl.ds(to, TAIL)]],
                           bufr0.at[pl.ds(0, TAIL)], semr0)
    cps = pltpu.async_copy(pres_hbm.at[rowv.at[pl.ds(to, TAIL)]],
                           bufs0.at[pl.ds(0, TAIL)], sems0)
    cpr.wait()
    cps.wait()
    transpose(bufr0, bufs0, buft0, TAIL // LANE)
    pltpu.sync_copy(buft0.at[:, pl.ds(0, TAIL)],
                    gt_hbm.at[:, pl.ds(ebase + to, TAIL)])


def _gather(prer_p, pres_p, col1, row1):
    mesh = plsc.VectorSubcoreMesh(core_axis_name="c", subcore_axis_name="s")
    fn = pl.kernel(
        _gather_body,
        compiler_params=pltpu.CompilerParams(use_tc_tiling_on_sc=False,
                                             needs_layout_passes=False),
        out_type=jax.ShapeDtypeStruct((LATENT, N_EDGES), jnp.float32),
        mesh=mesh,
        scratch_types=[
            pltpu.VMEM((EPW,), jnp.int32),
            pltpu.VMEM((EPW,), jnp.int32),
            pltpu.VMEM((GROUP_E, LATENT), jnp.float32),
            pltpu.VMEM((GROUP_E, LATENT), jnp.float32),
            pltpu.VMEM((GROUP_E, LATENT), jnp.float32),
            pltpu.VMEM((GROUP_E, LATENT), jnp.float32),
            pltpu.VMEM((LATENT, GROUP_E), jnp.float32),
            pltpu.VMEM((LATENT, GROUP_E), jnp.float32),
            pltpu.SemaphoreType.DMA,
            pltpu.SemaphoreType.DMA,
            pltpu.SemaphoreType.DMA,
            pltpu.SemaphoreType.DMA,
            pltpu.SemaphoreType.DMA,
            pltpu.SemaphoreType.DMA,
        ],
    )
    prer = prer_p.reshape(N_NODES, LATENT)
    pres = pres_p.reshape(N_NODES, LATENT)
    return fn(prer, pres, col1, row1)


_BLK = 32000


def _mlp_body(et_ref, gt_ref, w1t_ref, w2t_ref, bb_ref, out_ref):
    t = jnp.dot(w1t_ref[...], et_ref[...], preferred_element_type=jnp.float32)
    h = jnp.maximum(t + gt_ref[...] + bb_ref[:, 0:1], 0.0)
    h2 = jnp.dot(w2t_ref[...], h, preferred_element_type=jnp.float32) + bb_ref[:, 1:2]
    h2 = jnp.maximum(h2, 0.0)
    mu = jnp.mean(h2, axis=0, keepdims=True)
    d = h2 - mu
    var = jnp.mean(d * d, axis=0, keepdims=True)
    out_ref[...] = d * lax.rsqrt(var + 1e-5) * bb_ref[:, 2:3] + bb_ref[:, 3:4]


def _mlp(et, gt, w1t, w2t, bb):
    big = pl.BlockSpec((LATENT, _BLK), lambda i: (0, i))
    return pl.pallas_call(
        _mlp_body,
        grid=(N_EDGES // _BLK,),
        in_specs=[big, big,
                  pl.BlockSpec((LATENT, LATENT), lambda i: (0, 0)),
                  pl.BlockSpec((LATENT, LATENT), lambda i: (0, 0)),
                  pl.BlockSpec((LATENT, 4), lambda i: (0, 0))],
        out_specs=big,
        out_shape=jax.ShapeDtypeStruct((LATENT, N_EDGES), jnp.float32),
    )(et, gt, w1t, w2t, bb)


def kernel(x, e, u, edge_index, batch, W1, b1, W2, b2, gamma, beta):
    f32 = jnp.float32
    eye8 = jnp.eye(8, dtype=f32)
    w_r = W1[D_EDGE:D_EDGE + D_FEAT]
    w_s = W1[D_EDGE + D_FEAT:D_EDGE + 2 * D_FEAT]
    w_u = W1[D_EDGE + 2 * D_FEAT:]
    wbr = jnp.kron(eye8, w_r)
    wbs = jnp.kron(eye8, w_s)
    m8 = jnp.kron(eye8, jnp.ones((16, 16), f32))
    rmat = jnp.kron(eye8, jnp.ones((1, 16), f32))

    x8 = x.reshape(N_NODES // 8, 8 * D_FEAT)
    b8 = batch.reshape(N_NODES // 8, 8)
    prer_p, pres_p = _node_pre(x8, wbr, wbs, u, w_u, b8, rmat, m8)

    gt = _gather(prer_p, pres_p, edge_index[1], edge_index[0])

    w1t = W1[:D_EDGE].T
    w2t = W2.T
    bb = jnp.stack([b1, b2, gamma, beta], axis=1)

    outt = _mlp(e.T, gt, w1t, w2t, bb)
    return outt.T


# speedup vs baseline: 24.2399x; 1.1264x over previous
import functools

import jax
import jax.numpy as jnp
from jax import lax
from jax.experimental import pallas as pl
from jax.experimental.pallas import tpu as pltpu
from jax.experimental.pallas import tpu_sc as plsc

N_NODES = 10000
N_EDGES = 320000
N_GRAPHS = 16
D_FEAT = 128
D_EDGE = 16
LATENT = 16

NC = 2
NS = 16
NW = NC * NS
LANE = 16

EPW = N_EDGES // NW
STEP = 128
G = 3
NGRP = 26
NPAIR = NGRP // 2
GROUP_E = G * STEP
TAIL = EPW - NGRP * GROUP_E
VPG = GROUP_E // LANE


def _node_pre_body(x8_ref, wbr_ref, wbs_ref, u_ref, wu_ref, b8_ref, r_ref,
                   m8_ref, prer_ref, pres_ref):
    x8 = x8_ref[...]
    prer_ref[...] = jnp.dot(x8, wbr_ref[...], preferred_element_type=jnp.float32)
    uw = jnp.dot(u_ref[...], wu_ref[...], preferred_element_type=jnp.float32)
    uw8 = jnp.tile(uw, (8, 8)) * m8_ref[...]
    brep = jnp.dot(b8_ref[...].astype(jnp.float32), r_ref[...],
                   preferred_element_type=jnp.float32)
    g16 = (lax.broadcasted_iota(jnp.int32, (1, 128), 1) % 16).astype(jnp.float32)
    oh = (brep == g16).astype(jnp.float32)
    pres_ref[...] = (jnp.dot(x8, wbs_ref[...], preferred_element_type=jnp.float32)
                     + jnp.dot(oh, uw8, preferred_element_type=jnp.float32))


def _node_pre(x8, wbr, wbs, u, wu, b8, r, m8):
    return pl.pallas_call(
        _node_pre_body,
        out_shape=(
            jax.ShapeDtypeStruct((N_NODES // 8, 128), jnp.float32),
            jax.ShapeDtypeStruct((N_NODES // 8, 128), jnp.float32),
        ),
    )(x8, wbr, wbs, u, wu, b8, r, m8)


def _gather_body(prer_hbm, pres_hbm, col_hbm, row_hbm, ghi_hbm, glo_hbm,
                 colv, rowv, bufr0, bufs0, bufr1, bufs1,
                 bhi0, blo0, bhi1, blo1,
                 semr0, sems0, semr1, sems1, semw0, semw1):
    wid = lax.axis_index("s") * NC + lax.axis_index("c")
    ebase = wid * EPW
    pltpu.sync_copy(col_hbm.at[pl.ds(ebase, EPW)], colv)
    pltpu.sync_copy(row_hbm.at[pl.ds(ebase, EPW)], rowv)
    lane = lax.broadcasted_iota(jnp.int32, (LANE,), 0)

    def fire(g, bufr, bufs, semr, sems):
        for b in range(G):
            o = g * GROUP_E + b * STEP
            d = pl.ds(b * STEP, STEP)
            pltpu.async_copy(prer_hbm.at[colv.at[pl.ds(o, STEP)]],
                             bufr.at[d], semr)
            pltpu.async_copy(pres_hbm.at[rowv.at[pl.ds(o, STEP)]],
                             bufs.at[d], sems)

    def drain_gathers(bufr, bufs, semr, sems):
        idx0 = colv.at[pl.ds(0, STEP)]
        for b in range(G):
            d = pl.ds(b * STEP, STEP)
            pltpu.make_async_copy(prer_hbm.at[idx0], bufr.at[d], semr).wait()
            pltpu.make_async_copy(pres_hbm.at[idx0], bufs.at[d], sems).wait()

    def transpose(bufr, bufs, bhi, blo, n_vregs):
        def make_frow(half, fofs):
            def frow(f):
                fcol = jnp.full((LANE,), f + fofs, jnp.int32)
                vs = []
                for b in range(n_vregs):
                    ridx = lane + (LANE * b)
                    vs.append(plsc.load_gather(bufr, [ridx, fcol])
                              + plsc.load_gather(bufs, [ridx, fcol]))
                for b in range(n_vregs):
                    half[f, pl.ds(b * LANE, LANE)] = vs[b]
            return frow
        plsc.parallel_loop(0, 8)(make_frow(bhi, 0))
        plsc.parallel_loop(0, 8)(make_frow(blo, 8))

    def write(bhi, blo, g, semw):
        d = pl.ds(ebase + g * GROUP_E, GROUP_E)
        pltpu.async_copy(bhi, ghi_hbm.at[:, d], semw)
        pltpu.async_copy(blo, glo_hbm.at[:, d], semw)

    def drain_write(bhi, blo, semw):
        d = pl.ds(ebase, GROUP_E)
        pltpu.make_async_copy(bhi, ghi_hbm.at[:, d], semw).wait()
        pltpu.make_async_copy(blo, glo_hbm.at[:, d], semw).wait()

    fire(0, bufr0, bufs0, semr0, sems0)

    def pair(p, carry):
        g0 = 2 * p
        fire(g0 + 1, bufr1, bufs1, semr1, sems1)
        drain_gathers(bufr0, bufs0, semr0, sems0)

        @pl.when(p > 0)
        def _():
            drain_write(bhi0, blo0, semw0)

        transpose(bufr0, bufs0, bhi0, blo0, VPG)
        write(bhi0, blo0, g0, semw0)

        @pl.when(p < NPAIR - 1)
        def _():
            fire(g0 + 2, bufr0, bufs0, semr0, sems0)

        drain_gathers(bufr1, bufs1, semr1, sems1)

        @pl.when(p > 0)
        def _():
            drain_write(bhi1, blo1, semw1)

        transpose(bufr1, bufs1, bhi1, blo1, VPG)
        write(bhi1, blo1, g0 + 1, semw1)
        return carry

    lax.fori_loop(0, NPAIR, pair, 0)
    drain_write(bhi0, blo0, semw0)
    drain_write(bhi1, blo1, semw1)

    to = NGRP * GROUP_E
    cpr = pltpu.async_copy(prer_hbm.at[colv.at[pl.ds(to, TAIL)]],
                           bufr0.at[pl.ds(0, TAIL)], semr0)
    cps = pltpu.async_copy(pres_hbm.at[rowv.at[pl.ds(to, TAIL)]],
                           bufs0.at[pl.ds(0, TAIL)], sems0)
    cpr.wait()
    cps.wait()
    transpose(bufr0, bufs0, bhi0, blo0, TAIL // LANE)
    dtl = pl.ds(ebase + to, TAIL)
    pltpu.sync_copy(bhi0.at[:, pl.ds(0, TAIL)], ghi_hbm.at[:, dtl])
    pltpu.sync_copy(blo0.at[:, pl.ds(0, TAIL)], glo_hbm.at[:, dtl])


def _gather(prer_p, pres_p, col1, row1):
    mesh = plsc.VectorSubcoreMesh(core_axis_name="c", subcore_axis_name="s")
    fn = pl.kernel(
        _gather_body,
        compiler_params=pltpu.CompilerParams(use_tc_tiling_on_sc=False,
                                             needs_layout_passes=False),
        out_type=(
            jax.ShapeDtypeStruct((8, N_EDGES), jnp.float32),
            jax.ShapeDtypeStruct((8, N_EDGES), jnp.float32),
        ),
        mesh=mesh,
        scratch_types=[
            pltpu.VMEM((EPW,), jnp.int32),
            pltpu.VMEM((EPW,), jnp.int32),
            pltpu.VMEM((GROUP_E, LATENT), jnp.float32),
            pltpu.VMEM((GROUP_E, LATENT), jnp.float32),
            pltpu.VMEM((GROUP_E, LATENT), jnp.float32),
            pltpu.VMEM((GROUP_E, LATENT), jnp.float32),
            pltpu.VMEM((8, GROUP_E), jnp.float32),
            pltpu.VMEM((8, GROUP_E), jnp.float32),
            pltpu.VMEM((8, GROUP_E), jnp.float32),
            pltpu.VMEM((8, GROUP_E), jnp.float32),
            pltpu.SemaphoreType.DMA,
            pltpu.SemaphoreType.DMA,
            pltpu.SemaphoreType.DMA,
            pltpu.SemaphoreType.DMA,
            pltpu.SemaphoreType.DMA,
            pltpu.SemaphoreType.DMA,
        ],
    )
    prer = prer_p.reshape(N_NODES, LATENT)
    pres = pres_p.reshape(N_NODES, LATENT)
    return fn(prer, pres, col1, row1)


_BLK = 32000


def _mlp_body(et_ref, ghi_ref, glo_ref, w1t_ref, w2t_ref, bb_ref, out_ref):
    t = jnp.dot(w1t_ref[...], et_ref[...], preferred_element_type=jnp.float32)
    g = jnp.concatenate([ghi_ref[...], glo_ref[...]], axis=0)
    h = jnp.maximum(t + g + bb_ref[:, 0:1], 0.0)
    h2 = jnp.dot(w2t_ref[...], h, preferred_element_type=jnp.float32) + bb_ref[:, 1:2]
    h2 = jnp.maximum(h2, 0.0)
    mu = jnp.mean(h2, axis=0, keepdims=True)
    d = h2 - mu
    var = jnp.mean(d * d, axis=0, keepdims=True)
    out_ref[...] = d * lax.rsqrt(var + 1e-5) * bb_ref[:, 2:3] + bb_ref[:, 3:4]


def _mlp(et, ghi, glo, w1t, w2t, bb):
    big = pl.BlockSpec((LATENT, _BLK), lambda i: (0, i))
    half = pl.BlockSpec((8, _BLK), lambda i: (0, i))
    return pl.pallas_call(
        _mlp_body,
        grid=(N_EDGES // _BLK,),
        in_specs=[big, half, half,
                  pl.BlockSpec((LATENT, LATENT), lambda i: (0, 0)),
                  pl.BlockSpec((LATENT, LATENT), lambda i: (0, 0)),
                  pl.BlockSpec((LATENT, 4), lambda i: (0, 0))],
        out_specs=big,
        out_shape=jax.ShapeDtypeStruct((LATENT, N_EDGES), jnp.float32),
    )(et, ghi, glo, w1t, w2t, bb)


def kernel(x, e, u, edge_index, batch, W1, b1, W2, b2, gamma, beta):
    f32 = jnp.float32
    eye8 = jnp.eye(8, dtype=f32)
    w_r = W1[D_EDGE:D_EDGE + D_FEAT]
    w_s = W1[D_EDGE + D_FEAT:D_EDGE + 2 * D_FEAT]
    w_u = W1[D_EDGE + 2 * D_FEAT:]
    wbr = jnp.kron(eye8, w_r)
    wbs = jnp.kron(eye8, w_s)
    m8 = jnp.kron(eye8, jnp.ones((16, 16), f32))
    rmat = jnp.kron(eye8, jnp.ones((1, 16), f32))

    x8 = x.reshape(N_NODES // 8, 8 * D_FEAT)
    b8 = batch.reshape(N_NODES // 8, 8)
    prer_p, pres_p = _node_pre(x8, wbr, wbs, u, w_u, b8, rmat, m8)

    ghi, glo = _gather(prer_p, pres_p, edge_index[1], edge_index[0])

    w1t = W1[:D_EDGE].T
    w2t = W2.T
    bb = jnp.stack([b1, b2, gamma, beta], axis=1)

    outt = _mlp(e.T, ghi, glo, w1t, w2t, bb)
    return outt.T
